# baseline (speedup 1.0000x reference)
import functools

import jax
import jax.numpy as jnp
from jax.experimental import pallas as pl

_K = 32
_ROWS = 64


_SUB = 64
_LANES = 128
_PER_COL = 8


def _knn_body(t_ref, s_ref, idx_ref, val_ref):
    t = t_ref[0]
    s = s_ref[0]
    mm = jax.lax.dot_general(
        t, s, (((1,), (1,)), ((), ())), preferred_element_type=jnp.float32
    )
    r0 = jnp.sum(t * t, axis=1, keepdims=True)
    r1 = jnp.sum(s * s, axis=1)[None, :]
    d = (r0 - 2.0 * mm) + r1

    rows, ns = d.shape
    inf = jnp.float32(jnp.inf)

    v = d.reshape(rows, _SUB, _LANES)
    sub = jax.lax.broadcasted_iota(jnp.int32, (rows, _SUB, _LANES), 1)
    big_sub = jnp.int32(_SUB)
    c_vals, c_sub = [], []
    for _ in range(_PER_COL):
        m = jnp.min(v, axis=1, keepdims=True)
        eq = v == m
        im = jnp.min(jnp.where(eq, sub, big_sub), axis=1, keepdims=True)
        c_vals.append(m)
        c_sub.append(im)
        v = jnp.where(sub == im, inf, v)
    cv = jnp.concatenate(c_vals, axis=1)
    ci = jnp.concatenate(c_sub, axis=1)
    lane = jax.lax.broadcasted_iota(jnp.int32, (rows, _PER_COL, _LANES), 2)
    gi = ci * _LANES + lane

    big_i = jnp.int32(ns)
    vals, idxs = [], []
    for _ in range(_K):
        m = jnp.min(cv, axis=(1, 2), keepdims=True)
        eq = cv == m
        im = jnp.min(jnp.where(eq, gi, big_i), axis=(1, 2), keepdims=True)
        vals.append(m[:, 0])
        idxs.append(im[:, 0])
        cv = jnp.where(gi == im, inf, cv)
    val_ref[0] = jnp.concatenate(vals, axis=1)
    idx_ref[0] = jnp.concatenate(idxs, axis=1)


@functools.partial(jax.jit, static_argnames=("interpret",))
def _impl(source, target, interpret=False):
    b, nt, _ = target.shape
    ns = source.shape[1]
    grid = (b, nt // _ROWS)
    idx, vals = pl.pallas_call(
        _knn_body,
        grid=grid,
        in_specs=[
            pl.BlockSpec((1, _ROWS, 3), lambda bi, i: (bi, i, 0)),
            pl.BlockSpec((1, ns, 3), lambda bi, i: (bi, 0, 0)),
        ],
        out_specs=[
            pl.BlockSpec((1, _ROWS, _K), lambda bi, i: (bi, i, 0)),
            pl.BlockSpec((1, _ROWS, _K), lambda bi, i: (bi, i, 0)),
        ],
        out_shape=[
            jax.ShapeDtypeStruct((b, nt, _K), jnp.int32),
            jax.ShapeDtypeStruct((b, nt, _K), jnp.float32),
        ],
        interpret=interpret,
    )(target, source)
    batch_idx = jnp.broadcast_to(
        jnp.arange(b, dtype=jnp.int32).reshape(b, 1, 1), (b, nt, _K)
    )
    patches_idx = jnp.stack([batch_idx, idx], axis=-1)
    return patches_idx, vals


def kernel(source, target):
    return _impl(source, target)
